# baseline (device time: 46839 ns/iter reference)
import jax
import jax.numpy as jnp
from jax import lax
from jax.experimental import pallas as pl
from jax.experimental.pallas import tpu as pltpu

N_DEV = 4
B, SQ, SKV, HQ, DH = 2, 128, 512, 16, 64
D_MODEL = 512
H_LOC = HQ // N_DEV
HD_LOC = H_LOC * DH
SKV_LOC = SKV // N_DEV

_DeviceIdType = getattr(pl, "DeviceIdType", None) or pltpu.DeviceIdType
MESH = _DeviceIdType.MESH
_sem_signal = getattr(pl, "semaphore_signal", None) or pltpu.semaphore_signal
_sem_wait = getattr(pl, "semaphore_wait", None) or pltpu.semaphore_wait
_CompilerParams = getattr(pltpu, "CompilerParams", None) or pltpu.TPUCompilerParams


def kernel(x, Wq, K_ext, V_ext, Wo):
    k4 = jnp.transpose(K_ext.reshape(B, SKV_LOC, N_DEV, HD_LOC), (2, 0, 1, 3))
    v4 = jnp.transpose(V_ext.reshape(B, SKV_LOC, N_DEV, HD_LOC), (2, 0, 1, 3))

    def body(x_ref, wq_ref, k_ref, v_ref, wo_ref, out_ref,
             kb_ref, vb_ref, comm_ref,
             local_sems, send_sems, krecv_sems, vrecv_sems,
             ring_send_sems, ring_recv_sems):
        my = lax.axis_index("i")
        right = lax.rem(my + 1, N_DEV)

        barrier = pltpu.get_barrier_semaphore()
        for o in range(1, N_DEV):
            _sem_signal(barrier, inc=1, device_id=(lax.rem(my + o, N_DEV),),
                        device_id_type=MESH)
        _sem_wait(barrier, N_DEV - 1)

        kcp = pltpu.make_async_copy(k_ref.at[my], kb_ref.at[my], local_sems.at[0])
        vcp = pltpu.make_async_copy(v_ref.at[my], vb_ref.at[my], local_sems.at[1])
        kcp.start()
        vcp.start()

        sends = []
        for o in range(1, N_DEV):
            peer = lax.rem(my + o, N_DEV)
            kr = pltpu.make_async_remote_copy(
                src_ref=k_ref.at[peer], dst_ref=kb_ref.at[my],
                send_sem=send_sems.at[2 * (o - 1)], recv_sem=krecv_sems.at[my],
                device_id=(peer,), device_id_type=MESH)
            vr = pltpu.make_async_remote_copy(
                src_ref=v_ref.at[peer], dst_ref=vb_ref.at[my],
                send_sem=send_sems.at[2 * (o - 1) + 1], recv_sem=vrecv_sems.at[my],
                device_id=(peer,), device_id_type=MESH)
            kr.start()
            vr.start()
            sends += [kr, vr]

        q = [jnp.dot(x_ref[b], wq_ref[...], preferred_element_type=jnp.float32)
             for b in range(B)]

        kcp.wait()
        vcp.wait()
        for o in range(1, N_DEV):
            src = lax.rem(my + o, N_DEV)
            for buf, sems in ((kb_ref, krecv_sems), (vb_ref, vrecv_sems)):
                rr = pltpu.make_async_remote_copy(
                    src_ref=buf.at[src], dst_ref=buf.at[src],
                    send_sem=send_sems.at[0], recv_sem=sems.at[src],
                    device_id=(src,), device_id_type=MESH)
                rr.wait_recv()
        for s_ in sends:
            s_.wait_send()

        rb = lax.broadcasted_iota(jnp.int32, (SQ, SKV), 0) // 64
        cb = lax.broadcasted_iota(jnp.int32, (SQ, SKV), 1) // 64
        mask = (rb == cb) | (cb == 0) | (lax.rem(rb + cb, 3) == 0)
        neg = jnp.float32(-1e9)

        for b in range(B):
            ctx_parts = []
            for h in range(H_LOC):
                qh = q[b][:, h * DH:(h + 1) * DH]
                blocks = [
                    lax.dot_general(
                        qh, kb_ref[s, b][:, h * DH:(h + 1) * DH],
                        (((1,), (1,)), ((), ())),
                        preferred_element_type=jnp.float32)
                    for s in range(N_DEV)
                ]
                scores = jnp.concatenate(blocks, axis=1) * 0.125
                scores = jnp.where(mask, scores, neg)
                m = jnp.max(scores, axis=1, keepdims=True)
                w = jnp.exp(scores - m)
                w = w / jnp.sum(w, axis=1, keepdims=True)
                acc = None
                for s in range(N_DEV):
                    p = jnp.dot(w[:, s * SKV_LOC:(s + 1) * SKV_LOC],
                                vb_ref[s, b][:, h * DH:(h + 1) * DH],
                                preferred_element_type=jnp.float32)
                    acc = p if acc is None else acc + p
                ctx_parts.append(acc)
            ctx_b = jnp.concatenate(ctx_parts, axis=1)
            partial_b = jnp.dot(ctx_b, wo_ref[...],
                                preferred_element_type=jnp.float32)
            out_ref[b] = partial_b
            comm_ref[0, b] = partial_b

        for hop in range(N_DEV - 1):
            ss, rs = hop % 2, (hop + 1) % 2
            rdma = pltpu.make_async_remote_copy(
                src_ref=comm_ref.at[ss], dst_ref=comm_ref.at[rs],
                send_sem=ring_send_sems.at[ss], recv_sem=ring_recv_sems.at[rs],
                device_id=(right,), device_id_type=MESH)
            rdma.start()
            rdma.wait()
            for b in range(B):
                out_ref[b] = out_ref[b] + comm_ref[rs, b]

    return pl.pallas_call(
        body,
        out_shape=jax.ShapeDtypeStruct((B, SQ, D_MODEL), jnp.float32),
        in_specs=[pl.BlockSpec(memory_space=pltpu.VMEM)] * 5,
        out_specs=pl.BlockSpec(memory_space=pltpu.VMEM),
        scratch_shapes=[
            pltpu.VMEM((N_DEV, B, SKV_LOC, HD_LOC), jnp.float32),
            pltpu.VMEM((N_DEV, B, SKV_LOC, HD_LOC), jnp.float32),
            pltpu.VMEM((2, B, SQ, D_MODEL), jnp.float32),
            pltpu.SemaphoreType.DMA((2,)),
            pltpu.SemaphoreType.DMA((6,)),
            pltpu.SemaphoreType.DMA((N_DEV,)),
            pltpu.SemaphoreType.DMA((N_DEV,)),
            pltpu.SemaphoreType.DMA((2,)),
            pltpu.SemaphoreType.DMA((2,)),
        ],
        compiler_params=_CompilerParams(collective_id=0),
    )(x, Wq, k4, v4, Wo)


# device time: 37852 ns/iter; 1.2374x vs baseline; 1.2374x over previous
import jax
import jax.numpy as jnp
from jax import lax
from jax.experimental import pallas as pl
from jax.experimental.pallas import tpu as pltpu

N_DEV = 4
B, SQ, SKV, HQ, DH = 2, 128, 512, 16, 64
D_MODEL = 512
H_LOC = HQ // N_DEV
HD_LOC = H_LOC * DH
SKV_LOC = SKV // N_DEV

SRC_SLICES = {0: (0, 128, 0), 1: (0, 128, 128), 2: (64, 64, 256), 3: (0, 64, 320)}
NKV = 384

_DeviceIdType = getattr(pl, "DeviceIdType", None) or pltpu.DeviceIdType
MESH = _DeviceIdType.MESH
_sem_signal = getattr(pl, "semaphore_signal", None) or pltpu.semaphore_signal
_sem_wait = getattr(pl, "semaphore_wait", None) or pltpu.semaphore_wait
_CompilerParams = getattr(pltpu, "CompilerParams", None) or pltpu.TPUCompilerParams


def kernel(x, Wq, K_ext, V_ext, Wo):
    k2 = K_ext.reshape(B, SKV_LOC, HQ * DH)
    v2 = V_ext.reshape(B, SKV_LOC, HQ * DH)

    def body(x_ref, wq_ref, k_ref, v_ref, wo_ref, out_ref,
             kb_ref, vb_ref, ex_ref,
             local_sems, send_sems, krecv_sems, vrecv_sems,
             ex_send_sems, ex_recv_sems):
        my = lax.axis_index("i")

        barrier = pltpu.get_barrier_semaphore()
        for o in range(1, N_DEV):
            _sem_signal(barrier, inc=1, device_id=(lax.rem(my + o, N_DEV),),
                        device_id_type=MESH)
        _sem_wait(barrier, N_DEV - 1)

        def kv_rdma(src_dev, peer, ref, buf, sem, rsems):
            lo, ln, dst = SRC_SLICES[src_dev]
            return pltpu.make_async_remote_copy(
                src_ref=ref.at[:, lo:lo + ln, peer * HD_LOC:(peer + 1) * HD_LOC],
                dst_ref=buf.at[:, dst:dst + ln, :],
                send_sem=sem, recv_sem=rsems.at[src_dev],
                device_id=(peer,), device_id_type=MESH)

        for s in range(N_DEV):
            @pl.when(my == s)
            def _(s=s):
                lo, ln, dst = SRC_SLICES[s]
                pltpu.make_async_copy(
                    k_ref.at[:, lo:lo + ln, s * HD_LOC:(s + 1) * HD_LOC],
                    kb_ref.at[:, dst:dst + ln, :], local_sems.at[0]).start()
                pltpu.make_async_copy(
                    v_ref.at[:, lo:lo + ln, s * HD_LOC:(s + 1) * HD_LOC],
                    vb_ref.at[:, dst:dst + ln, :], local_sems.at[1]).start()
                for i, p in enumerate([p for p in range(N_DEV) if p != s]):
                    kv_rdma(s, p, k_ref, kb_ref, send_sems.at[2 * i],
                            krecv_sems).start()
                    kv_rdma(s, p, v_ref, vb_ref, send_sems.at[2 * i + 1],
                            vrecv_sems).start()

        q = [jnp.dot(x_ref[b], wq_ref[...], preferred_element_type=jnp.float32)
             for b in range(B)]

        for s in range(N_DEV):
            @pl.when(my == s)
            def _(s=s):
                lo, ln, dst = SRC_SLICES[s]
                pltpu.make_async_copy(
                    k_ref.at[:, lo:lo + ln, s * HD_LOC:(s + 1) * HD_LOC],
                    kb_ref.at[:, dst:dst + ln, :], local_sems.at[0]).wait()
                pltpu.make_async_copy(
                    v_ref.at[:, lo:lo + ln, s * HD_LOC:(s + 1) * HD_LOC],
                    vb_ref.at[:, dst:dst + ln, :], local_sems.at[1]).wait()
                for r in [r for r in range(N_DEV) if r != s]:
                    rlo, rln, rdst = SRC_SLICES[r]
                    for buf, rsems in ((kb_ref, krecv_sems), (vb_ref, vrecv_sems)):
                        pltpu.make_async_remote_copy(
                            src_ref=buf.at[:, rdst:rdst + rln, :],
                            dst_ref=buf.at[:, rdst:rdst + rln, :],
                            send_sem=send_sems.at[0], recv_sem=rsems.at[r],
                            device_id=(r,), device_id_type=MESH).wait_recv()
                for i, p in enumerate([p for p in range(N_DEV) if p != s]):
                    kv_rdma(s, p, k_ref, kb_ref, send_sems.at[2 * i],
                            krecv_sems).wait_send()
                    kv_rdma(s, p, v_ref, vb_ref, send_sems.at[2 * i + 1],
                            vrecv_sems).wait_send()

        rb = lax.broadcasted_iota(jnp.int32, (SQ, NKV), 0) // 64
        c6 = lax.broadcasted_iota(jnp.int32, (SQ, NKV), 1) // 64
        cb = c6 + (c6 >= 4).astype(jnp.int32)
        mask = (rb == cb) | (cb == 0) | (lax.rem(rb + cb, 3) == 0)
        neg = jnp.float32(-1e9)

        for b in range(B):
            ctx_parts = []
            for h in range(H_LOC):
                qh = q[b][:, h * DH:(h + 1) * DH]
                scores = lax.dot_general(
                    qh, kb_ref[b][:, h * DH:(h + 1) * DH],
                    (((1,), (1,)), ((), ())),
                    preferred_element_type=jnp.float32) * 0.125
                scores = jnp.where(mask, scores, neg)
                m = jnp.max(scores, axis=1, keepdims=True)
                w = jnp.exp(scores - m)
                w = w / jnp.sum(w, axis=1, keepdims=True)
                ctx_parts.append(
                    jnp.dot(w, vb_ref[b][:, h * DH:(h + 1) * DH],
                            preferred_element_type=jnp.float32))
            ctx_b = jnp.concatenate(ctx_parts, axis=1)
            out_ref[b] = jnp.dot(ctx_b, wo_ref[...],
                                 preferred_element_type=jnp.float32)

        p1 = my + 1 - 2 * lax.rem(my, 2)
        p2 = 3 - my
        for r, partner in enumerate([p1, p2]):
            ex = pltpu.make_async_remote_copy(
                src_ref=out_ref, dst_ref=ex_ref.at[r],
                send_sem=ex_send_sems.at[r], recv_sem=ex_recv_sems.at[r],
                device_id=(partner,), device_id_type=MESH)
            ex.start()
            ex.wait()
            for b in range(B):
                out_ref[b] = out_ref[b] + ex_ref[r, b]

    return pl.pallas_call(
        body,
        out_shape=jax.ShapeDtypeStruct((B, SQ, D_MODEL), jnp.float32),
        in_specs=[pl.BlockSpec(memory_space=pltpu.VMEM)] * 5,
        out_specs=pl.BlockSpec(memory_space=pltpu.VMEM),
        scratch_shapes=[
            pltpu.VMEM((B, NKV, HD_LOC), jnp.float32),
            pltpu.VMEM((B, NKV, HD_LOC), jnp.float32),
            pltpu.VMEM((2, B, SQ, D_MODEL), jnp.float32),
            pltpu.SemaphoreType.DMA((2,)),
            pltpu.SemaphoreType.DMA((6,)),
            pltpu.SemaphoreType.DMA((N_DEV,)),
            pltpu.SemaphoreType.DMA((N_DEV,)),
            pltpu.SemaphoreType.DMA((2,)),
            pltpu.SemaphoreType.DMA((2,)),
        ],
        compiler_params=_CompilerParams(collective_id=0),
    )(x, Wq, k2, v2, Wo)


# device time: 31924 ns/iter; 1.4672x vs baseline; 1.1857x over previous
import jax
import jax.numpy as jnp
from jax import lax
from jax.experimental import pallas as pl
from jax.experimental.pallas import tpu as pltpu

N_DEV = 4
B, SQ, SKV, HQ, DH = 2, 128, 512, 16, 64
D_MODEL = 512
H_LOC = HQ // N_DEV
HD_LOC = H_LOC * DH
SKV_LOC = SKV // N_DEV

SRC_SLICES = {0: (0, 128, 0), 1: (0, 128, 128), 2: (64, 64, 256), 3: (0, 64, 320)}
NKV = 384

_DeviceIdType = getattr(pl, "DeviceIdType", None) or pltpu.DeviceIdType
MESH = _DeviceIdType.MESH
_sem_signal = getattr(pl, "semaphore_signal", None) or pltpu.semaphore_signal
_sem_wait = getattr(pl, "semaphore_wait", None) or pltpu.semaphore_wait
_CompilerParams = getattr(pltpu, "CompilerParams", None) or pltpu.TPUCompilerParams


def kernel(x, Wq, K_ext, V_ext, Wo):
    k2 = K_ext.reshape(B, SKV_LOC, HQ * DH)
    v2 = V_ext.reshape(B, SKV_LOC, HQ * DH)

    def body(x_ref, wq_ref, k_ref, v_ref, wo_ref, out_ref,
             kb_ref, vb_ref, ex_ref,
             local_sems, send_sems, krecv_sems, vrecv_sems,
             ex_send_sems, ex_recv_sems):
        my = lax.axis_index("i")

        barrier = pltpu.get_barrier_semaphore()
        for o in range(1, N_DEV):
            _sem_signal(barrier, inc=1, device_id=(lax.rem(my + o, N_DEV),),
                        device_id_type=MESH)
        _sem_wait(barrier, N_DEV - 1)

        def kv_rdma(src_dev, peer, ref, buf, sem, rsems):
            lo, ln, dst = SRC_SLICES[src_dev]
            return pltpu.make_async_remote_copy(
                src_ref=ref.at[:, lo:lo + ln, peer * HD_LOC:(peer + 1) * HD_LOC],
                dst_ref=buf.at[:, dst:dst + ln, :],
                send_sem=sem, recv_sem=rsems.at[src_dev],
                device_id=(peer,), device_id_type=MESH)

        def recv_wait(r, buf, rsems):
            rlo, rln, rdst = SRC_SLICES[r]
            pltpu.make_async_remote_copy(
                src_ref=buf.at[:, rdst:rdst + rln, :],
                dst_ref=buf.at[:, rdst:rdst + rln, :],
                send_sem=send_sems.at[0], recv_sem=rsems.at[r],
                device_id=(r,), device_id_type=MESH).wait_recv()

        for s in range(N_DEV):
            @pl.when(my == s)
            def _(s=s):
                lo, ln, dst = SRC_SLICES[s]
                peers = [p for p in range(N_DEV) if p != s]
                for i, p in enumerate(peers):
                    kv_rdma(s, p, k_ref, kb_ref, send_sems.at[2 * i],
                            krecv_sems).start()
                pltpu.make_async_copy(
                    k_ref.at[:, lo:lo + ln, s * HD_LOC:(s + 1) * HD_LOC],
                    kb_ref.at[:, dst:dst + ln, :], local_sems.at[0]).start()
                for i, p in enumerate(peers):
                    kv_rdma(s, p, v_ref, vb_ref, send_sems.at[2 * i + 1],
                            vrecv_sems).start()
                pltpu.make_async_copy(
                    v_ref.at[:, lo:lo + ln, s * HD_LOC:(s + 1) * HD_LOC],
                    vb_ref.at[:, dst:dst + ln, :], local_sems.at[1]).start()

        q = [jnp.dot(x_ref[b], wq_ref[...], preferred_element_type=jnp.float32)
             for b in range(B)]

        for s in range(N_DEV):
            @pl.when(my == s)
            def _(s=s):
                lo, ln, dst = SRC_SLICES[s]
                pltpu.make_async_copy(
                    k_ref.at[:, lo:lo + ln, s * HD_LOC:(s + 1) * HD_LOC],
                    kb_ref.at[:, dst:dst + ln, :], local_sems.at[0]).wait()
                for r in range(N_DEV):
                    if r != s:
                        recv_wait(r, kb_ref, krecv_sems)

        rb = lax.broadcasted_iota(jnp.int32, (SQ, NKV), 0) // 64
        c6 = lax.broadcasted_iota(jnp.int32, (SQ, NKV), 1) // 64
        cb = c6 + (c6 >= 4).astype(jnp.int32)
        mask = (rb == cb) | (cb == 0) | (lax.rem(rb + cb, 3) == 0)
        neg = jnp.float32(-1e9)

        w_all = []
        for b in range(B):
            w_b = []
            for h in range(H_LOC):
                qh = q[b][:, h * DH:(h + 1) * DH]
                scores = lax.dot_general(
                    qh, kb_ref[b][:, h * DH:(h + 1) * DH],
                    (((1,), (1,)), ((), ())),
                    preferred_element_type=jnp.float32) * 0.125
                scores = jnp.where(mask, scores, neg)
                m = jnp.max(scores, axis=1, keepdims=True)
                ew = jnp.exp(scores - m)
                w_b.append(ew / jnp.sum(ew, axis=1, keepdims=True))
            w_all.append(w_b)

        for s in range(N_DEV):
            @pl.when(my == s)
            def _(s=s):
                lo, ln, dst = SRC_SLICES[s]
                pltpu.make_async_copy(
                    v_ref.at[:, lo:lo + ln, s * HD_LOC:(s + 1) * HD_LOC],
                    vb_ref.at[:, dst:dst + ln, :], local_sems.at[1]).wait()
                for r in range(N_DEV):
                    if r != s:
                        recv_wait(r, vb_ref, vrecv_sems)

        partners = [my + 1 - 2 * lax.rem(my, 2), 3 - my]

        def exch(r, b):
            return pltpu.make_async_remote_copy(
                src_ref=out_ref.at[b], dst_ref=ex_ref.at[r, b],
                send_sem=ex_send_sems.at[2 * r + b],
                recv_sem=ex_recv_sems.at[2 * r + b],
                device_id=(partners[r],), device_id_type=MESH)

        e = {}
        for b in range(B):
            ctx_b = jnp.concatenate(
                [jnp.dot(w_all[b][h], vb_ref[b][:, h * DH:(h + 1) * DH],
                         preferred_element_type=jnp.float32)
                 for h in range(H_LOC)], axis=1)
            out_ref[b] = jnp.dot(ctx_b, wo_ref[...],
                                 preferred_element_type=jnp.float32)
            e[(0, b)] = exch(0, b)
            e[(0, b)].start()
        for b in range(B):
            e[(0, b)].wait()
            out_ref[b] = out_ref[b] + ex_ref[0, b]
            e[(1, b)] = exch(1, b)
            e[(1, b)].start()
        for b in range(B):
            e[(1, b)].wait()
            out_ref[b] = out_ref[b] + ex_ref[1, b]

        for s in range(N_DEV):
            @pl.when(my == s)
            def _(s=s):
                for i, p in enumerate([p for p in range(N_DEV) if p != s]):
                    kv_rdma(s, p, k_ref, kb_ref, send_sems.at[2 * i],
                            krecv_sems).wait_send()
                    kv_rdma(s, p, v_ref, vb_ref, send_sems.at[2 * i + 1],
                            vrecv_sems).wait_send()

    return pl.pallas_call(
        body,
        out_shape=jax.ShapeDtypeStruct((B, SQ, D_MODEL), jnp.float32),
        in_specs=[pl.BlockSpec(memory_space=pltpu.VMEM)] * 5,
        out_specs=pl.BlockSpec(memory_space=pltpu.VMEM),
        scratch_shapes=[
            pltpu.VMEM((B, NKV, HD_LOC), jnp.float32),
            pltpu.VMEM((B, NKV, HD_LOC), jnp.float32),
            pltpu.VMEM((2, B, SQ, D_MODEL), jnp.float32),
            pltpu.SemaphoreType.DMA((2,)),
            pltpu.SemaphoreType.DMA((6,)),
            pltpu.SemaphoreType.DMA((N_DEV,)),
            pltpu.SemaphoreType.DMA((N_DEV,)),
            pltpu.SemaphoreType.DMA((4,)),
            pltpu.SemaphoreType.DMA((4,)),
        ],
        compiler_params=_CompilerParams(collective_id=0),
    )(x, Wq, k2, v2, Wo)


# device time: 28668 ns/iter; 1.6338x vs baseline; 1.1136x over previous
import jax
import jax.numpy as jnp
from jax import lax
from jax.experimental import pallas as pl
from jax.experimental.pallas import tpu as pltpu

N_DEV = 4
B, SQ, SKV, HQ, DH = 2, 128, 512, 16, 64
D_MODEL = 512
H_LOC = HQ // N_DEV
HD_LOC = H_LOC * DH
SKV_LOC = SKV // N_DEV

SRC_SLICES = {0: (0, 128, 0), 1: (0, 128, 128), 2: (64, 64, 256), 3: (0, 64, 320)}
NKV = 384

_DeviceIdType = getattr(pl, "DeviceIdType", None) or pltpu.DeviceIdType
MESH = _DeviceIdType.MESH
_sem_signal = getattr(pl, "semaphore_signal", None) or pltpu.semaphore_signal
_sem_wait = getattr(pl, "semaphore_wait", None) or pltpu.semaphore_wait
_CompilerParams = getattr(pltpu, "CompilerParams", None) or pltpu.TPUCompilerParams


def kernel(x, Wq, K_ext, V_ext, Wo):
    k2 = K_ext.reshape(B, SKV_LOC, HQ * DH)
    v2 = V_ext.reshape(B, SKV_LOC, HQ * DH)

    def body(x_ref, wq_ref, k_ref, v_ref, wo_ref, out_ref,
             kb_ref, vb_ref, ex_ref,
             local_sems, send_sems, krecv_sems, vrecv_sems,
             ex_send_sems, ex_recv_sems):
        my = lax.axis_index("i")

        barrier = pltpu.get_barrier_semaphore()
        for o in range(1, N_DEV):
            _sem_signal(barrier, inc=1, device_id=(lax.rem(my + o, N_DEV),),
                        device_id_type=MESH)
        _sem_wait(barrier, N_DEV - 1)

        def kv_rdma(src_dev, peer, ref, buf, sem, rsems):
            lo, ln, dst = SRC_SLICES[src_dev]
            return pltpu.make_async_remote_copy(
                src_ref=ref.at[:, lo:lo + ln, peer * HD_LOC:(peer + 1) * HD_LOC],
                dst_ref=buf.at[:, dst:dst + ln, :],
                send_sem=sem, recv_sem=rsems.at[src_dev],
                device_id=(peer,), device_id_type=MESH)

        def recv_wait(r, buf, rsems):
            rlo, rln, rdst = SRC_SLICES[r]
            pltpu.make_async_remote_copy(
                src_ref=buf.at[:, rdst:rdst + rln, :],
                dst_ref=buf.at[:, rdst:rdst + rln, :],
                send_sem=send_sems.at[0], recv_sem=rsems.at[r],
                device_id=(r,), device_id_type=MESH).wait_recv()

        for s in range(N_DEV):
            @pl.when(my == s)
            def _(s=s):
                lo, ln, dst = SRC_SLICES[s]
                peers = [p for p in range(N_DEV) if p != s]
                for i, p in enumerate(peers):
                    kv_rdma(s, p, k_ref, kb_ref, send_sems.at[2 * i],
                            krecv_sems).start()
                pltpu.make_async_copy(
                    k_ref.at[:, lo:lo + ln, s * HD_LOC:(s + 1) * HD_LOC],
                    kb_ref.at[:, dst:dst + ln, :], local_sems.at[0]).start()
                for i, p in enumerate(peers):
                    kv_rdma(s, p, v_ref, vb_ref, send_sems.at[2 * i + 1],
                            vrecv_sems).start()
                pltpu.make_async_copy(
                    v_ref.at[:, lo:lo + ln, s * HD_LOC:(s + 1) * HD_LOC],
                    vb_ref.at[:, dst:dst + ln, :], local_sems.at[1]).start()

        wqs = wq_ref[...] * jnp.float32(0.125)
        q = [jnp.dot(x_ref[b], wqs, preferred_element_type=jnp.float32)
             for b in range(B)]

        for s in range(N_DEV):
            @pl.when(my == s)
            def _(s=s):
                lo, ln, dst = SRC_SLICES[s]
                pltpu.make_async_copy(
                    k_ref.at[:, lo:lo + ln, s * HD_LOC:(s + 1) * HD_LOC],
                    kb_ref.at[:, dst:dst + ln, :], local_sems.at[0]).wait()
                for r in range(N_DEV):
                    if r != s:
                        recv_wait(r, kb_ref, krecv_sems)

        rb = lax.broadcasted_iota(jnp.int32, (SQ, NKV), 0) // 64
        c6 = lax.broadcasted_iota(jnp.int32, (SQ, NKV), 1) // 64
        cb = c6 + (c6 >= 4).astype(jnp.int32)
        mask = (rb == cb) | (cb == 0) | (lax.rem(rb + cb, 3) == 0)

        w_all = []
        for b in range(B):
            w_b = []
            for h in range(H_LOC):
                qh = q[b][:, h * DH:(h + 1) * DH]
                scores = lax.dot_general(
                    qh, kb_ref[b][:, h * DH:(h + 1) * DH],
                    (((1,), (1,)), ((), ())),
                    preferred_element_type=jnp.float32)
                ew = jnp.where(mask, jnp.exp(scores), jnp.float32(0.0))
                w_b.append(ew / jnp.sum(ew, axis=1, keepdims=True))
            w_all.append(w_b)

        for s in range(N_DEV):
            @pl.when(my == s)
            def _(s=s):
                lo, ln, dst = SRC_SLICES[s]
                pltpu.make_async_copy(
                    v_ref.at[:, lo:lo + ln, s * HD_LOC:(s + 1) * HD_LOC],
                    vb_ref.at[:, dst:dst + ln, :], local_sems.at[1]).wait()
                for r in range(N_DEV):
                    if r != s:
                        recv_wait(r, vb_ref, vrecv_sems)

        partners = [my + 1 - 2 * lax.rem(my, 2), 3 - my]
        H2 = D_MODEL // 2

        def exch(r, b, half):
            sl = slice(half * H2, (half + 1) * H2)
            idx = 4 * r + 2 * b + half
            return pltpu.make_async_remote_copy(
                src_ref=out_ref.at[b, :, sl], dst_ref=ex_ref.at[r, b, :, sl],
                send_sem=ex_send_sems.at[idx], recv_sem=ex_recv_sems.at[idx],
                device_id=(partners[(r + half) % 2],),
                device_id_type=MESH)

        e = {}
        for b in range(B):
            ctx_b = jnp.concatenate(
                [jnp.dot(w_all[b][h], vb_ref[b][:, h * DH:(h + 1) * DH],
                         preferred_element_type=jnp.float32)
                 for h in range(H_LOC)], axis=1)
            out_ref[b] = jnp.dot(ctx_b, wo_ref[...],
                                 preferred_element_type=jnp.float32)
            for half in range(2):
                e[(0, b, half)] = exch(0, b, half)
                e[(0, b, half)].start()
        for b in range(B):
            e[(0, b, 0)].wait()
            e[(0, b, 1)].wait()
            out_ref[b] = out_ref[b] + ex_ref[0, b]
            for half in range(2):
                e[(1, b, half)] = exch(1, b, half)
                e[(1, b, half)].start()
        for b in range(B):
            e[(1, b, 0)].wait()
            e[(1, b, 1)].wait()
            out_ref[b] = out_ref[b] + ex_ref[1, b]

        for s in range(N_DEV):
            @pl.when(my == s)
            def _(s=s):
                for i, p in enumerate([p for p in range(N_DEV) if p != s]):
                    kv_rdma(s, p, k_ref, kb_ref, send_sems.at[2 * i],
                            krecv_sems).wait_send()
                    kv_rdma(s, p, v_ref, vb_ref, send_sems.at[2 * i + 1],
                            vrecv_sems).wait_send()

    return pl.pallas_call(
        body,
        out_shape=jax.ShapeDtypeStruct((B, SQ, D_MODEL), jnp.float32),
        in_specs=[pl.BlockSpec(memory_space=pltpu.VMEM)] * 5,
        out_specs=pl.BlockSpec(memory_space=pltpu.VMEM),
        scratch_shapes=[
            pltpu.VMEM((B, NKV, HD_LOC), jnp.float32),
            pltpu.VMEM((B, NKV, HD_LOC), jnp.float32),
            pltpu.VMEM((2, B, SQ, D_MODEL), jnp.float32),
            pltpu.SemaphoreType.DMA((2,)),
            pltpu.SemaphoreType.DMA((6,)),
            pltpu.SemaphoreType.DMA((N_DEV,)),
            pltpu.SemaphoreType.DMA((N_DEV,)),
            pltpu.SemaphoreType.DMA((8,)),
            pltpu.SemaphoreType.DMA((8,)),
        ],
        compiler_params=_CompilerParams(collective_id=0),
    )(x, Wq, k2, v2, Wo)
